# baseline (speedup 1.0000x reference)
import functools

import jax
import jax.numpy as jnp
from jax import lax
from jax.experimental import pallas as pl
from jax.experimental.pallas import tpu as pltpu
from jax.experimental.pallas import tpu_sc as plsc

_B = 1024
_S = 50
_D = 1000
_STRIPE = 128
_Q = 256
_QLAST = _D - 3 * _Q

_mesh = plsc.VectorSubcoreMesh(core_axis_name="c", subcore_axis_name="s")


@functools.partial(
    pl.kernel,
    mesh=_mesh,
    out_type=jax.ShapeDtypeStruct((_D, _B), jnp.float32),
    compiler_params=pltpu.CompilerParams(needs_layout_passes=False),
    scratch_types=[
        pltpu.VMEM((_S, _STRIPE), jnp.float32),
        pltpu.VMEM((_Q, _STRIPE), jnp.float32),
        pltpu.SemaphoreType.DMA,
    ],
)
def _hist_kernel(in_hbm, out_hbm, in_v, out_v, isem):
    wid = lax.axis_index("s") * 2 + lax.axis_index("c")
    stripe_base = (wid // 4) * _STRIPE
    q = wid % 4

    idma = pltpu.async_copy(
        in_hbm.at[:, pl.ds(stripe_base, _STRIPE)], in_v, isem
    )

    zeros = jnp.zeros((16,), jnp.float32)

    @plsc.parallel_loop(0, _Q, step=4, unroll=2)
    def _(i):
        r = pl.multiple_of(i, 4)
        for k in range(4):
            for c in range(0, _STRIPE, 16):
                out_v[r + k, pl.ds(c, 16)] = zeros

    idma.wait()

    lanes = lax.iota(jnp.int32, 16)
    ones = jnp.ones((16,), jnp.float32)

    @plsc.parallel_loop(0, _S, step=1, unroll=4)
    def _(c):
        for g in range(_STRIPE // 16):
            rows = lanes + g * 16
            vals = in_v[c, pl.ds(g * 16, 16)].astype(jnp.int32)
            mask = lax.shift_right_logical(vals, 8) == q
            local = lax.bitwise_and(vals, _Q - 1)
            plsc.addupdate_scatter(out_v, [local, rows], ones, mask=mask)

    @pl.when(q < 3)
    def _():
        pltpu.sync_copy(
            out_v, out_hbm.at[pl.ds(q * _Q, _Q), pl.ds(stripe_base, _STRIPE)]
        )

    @pl.when(q == 3)
    def _():
        pltpu.sync_copy(
            out_v.at[pl.ds(0, _QLAST)],
            out_hbm.at[pl.ds(3 * _Q, _QLAST), pl.ds(stripe_base, _STRIPE)],
        )


def kernel(inputs):
    return _hist_kernel(inputs.T).T


# speedup vs baseline: 1.0259x; 1.0259x over previous
import functools

import jax
import jax.numpy as jnp
from jax import lax
from jax.experimental import pallas as pl
from jax.experimental.pallas import tpu as pltpu
from jax.experimental.pallas import tpu_sc as plsc

_B = 1024
_S = 50
_D = 1000
_STRIPE = 128
_Q = 256
_QLAST = _D - 3 * _Q

_mesh = plsc.VectorSubcoreMesh(core_axis_name="c", subcore_axis_name="s")


@functools.partial(
    pl.kernel,
    mesh=_mesh,
    out_type=jax.ShapeDtypeStruct((_D, _B), jnp.float32),
    compiler_params=pltpu.CompilerParams(
        needs_layout_passes=False,
        skip_device_barrier=True,
        disable_semaphore_checks=True,
    ),
    scratch_types=[
        pltpu.VMEM((_S, _STRIPE), jnp.float32),
        pltpu.VMEM((_Q, _STRIPE), jnp.float32),
        pltpu.SemaphoreType.DMA,
    ],
)
def _hist_kernel(in_hbm, out_hbm, in_v, out_v, isem):
    wid = lax.axis_index("s") * 2 + lax.axis_index("c")
    stripe_base = (wid // 4) * _STRIPE
    q = wid % 4

    idma = pltpu.async_copy(
        in_hbm.at[:, pl.ds(stripe_base, _STRIPE)], in_v, isem
    )

    zeros = jnp.zeros((16,), jnp.float32)

    @plsc.parallel_loop(0, _Q, step=4)
    def _(i):
        r = pl.multiple_of(i, 4)
        for k in range(4):
            for c in range(0, _STRIPE, 16):
                out_v[r + k, pl.ds(c, 16)] = zeros

    idma.wait()

    lanes = lax.iota(jnp.int32, 16)
    ones = jnp.ones((16,), jnp.float32)

    @plsc.parallel_loop(0, _S, step=1, unroll=2)
    def _(c):
        for g in range(_STRIPE // 16):
            rows = lanes + g * 16
            vals = in_v[c, pl.ds(g * 16, 16)].astype(jnp.int32)
            mask = lax.shift_right_logical(vals, 8) == q
            local = lax.bitwise_and(vals, _Q - 1)
            plsc.addupdate_scatter(out_v, [local, rows], ones, mask=mask)

    @pl.when(q < 3)
    def _():
        pltpu.sync_copy(
            out_v, out_hbm.at[pl.ds(q * _Q, _Q), pl.ds(stripe_base, _STRIPE)]
        )

    @pl.when(q == 3)
    def _():
        pltpu.sync_copy(
            out_v.at[pl.ds(0, _QLAST)],
            out_hbm.at[pl.ds(3 * _Q, _QLAST), pl.ds(stripe_base, _STRIPE)],
        )


def kernel(inputs):
    return _hist_kernel(inputs.T).T


# speedup vs baseline: 1.0267x; 1.0009x over previous
import functools

import jax
import jax.numpy as jnp
from jax import lax
from jax.experimental import pallas as pl
from jax.experimental.pallas import tpu as pltpu
from jax.experimental.pallas import tpu_sc as plsc

_B = 1024
_S = 50
_D = 1000
_STRIPE = 128
_Q = 256
_QLAST = _D - 3 * _Q

_mesh = plsc.VectorSubcoreMesh(core_axis_name="c", subcore_axis_name="s")


@functools.partial(
    pl.kernel,
    mesh=_mesh,
    out_type=jax.ShapeDtypeStruct((_D, _B), jnp.float32),
    compiler_params=pltpu.CompilerParams(needs_layout_passes=False),
    scratch_types=[
        pltpu.VMEM((_S, _STRIPE), jnp.float32),
        pltpu.VMEM((_Q, _STRIPE), jnp.float32),
        pltpu.SemaphoreType.DMA,
    ],
)
def _hist_kernel(in_hbm, out_hbm, in_v, out_v, isem):
    wid = lax.axis_index("s") * 2 + lax.axis_index("c")
    stripe_base = (wid // 4) * _STRIPE
    q = wid % 4

    idma = pltpu.async_copy(
        in_hbm.at[:, pl.ds(stripe_base, _STRIPE)], in_v, isem
    )

    zeros = jnp.zeros((16,), jnp.float32)

    @plsc.parallel_loop(0, _Q, step=4)
    def _(i):
        r = pl.multiple_of(i, 4)
        for k in range(4):
            for c in range(0, _STRIPE, 16):
                out_v[r + k, pl.ds(c, 16)] = zeros

    idma.wait()

    lanes = lax.iota(jnp.int32, 16)
    ones = jnp.ones((16,), jnp.float32)

    @plsc.parallel_loop(0, _S, step=1, unroll=2)
    def _(c):
        for g in range(_STRIPE // 16):
            rows = lanes + g * 16
            vals = in_v[c, pl.ds(g * 16, 16)].astype(jnp.int32)
            mask = lax.shift_right_logical(vals, 8) == q
            local = lax.bitwise_and(vals, _Q - 1)
            plsc.addupdate_scatter(out_v, [local, rows], ones, mask=mask)

    @pl.when(q < 3)
    def _():
        pltpu.sync_copy(
            out_v, out_hbm.at[pl.ds(q * _Q, _Q), pl.ds(stripe_base, _STRIPE)]
        )

    @pl.when(q == 3)
    def _():
        pltpu.sync_copy(
            out_v.at[pl.ds(0, _QLAST)],
            out_hbm.at[pl.ds(3 * _Q, _QLAST), pl.ds(stripe_base, _STRIPE)],
        )


def kernel(inputs):
    return _hist_kernel(inputs.T).T
